# baseline (speedup 1.0000x reference)
import functools

import jax
import jax.numpy as jnp
from jax import lax
from jax.experimental import pallas as pl
from jax.experimental.pallas import tpu as pltpu
from jax.experimental.pallas import tpu_sc as plsc

_D = 300
_DP = 384
_B = 16384 * 200

_info = plsc.get_sparse_core_info()
_NW = _info.num_cores * _info.num_subcores
_RPW = _B // _NW
_C = 128
_STEPS = _RPW // _C

_mesh = plsc.VectorSubcoreMesh(core_axis_name="c", subcore_axis_name="s")


@functools.partial(
    pl.kernel,
    out_type=jax.ShapeDtypeStruct((_B, _DP), jnp.float32),
    mesh=_mesh,
    scratch_types=[
        pltpu.VMEM((_C,), jnp.int32),
        pltpu.VMEM((_C, _DP), jnp.float32),
        pltpu.SemaphoreType.DMA,
    ],
    compiler_params=pltpu.CompilerParams(use_tc_tiling_on_sc=False),
)
def _gather_kernel(x_hbm, table_hbm, out_hbm, idx_v, rows_v, sem):
    wid = lax.axis_index("s") * _info.num_cores + lax.axis_index("c")
    base = wid * _RPW

    @pl.loop(0, _STEPS)
    def _step(i):
        off = base + i * _C
        pltpu.sync_copy(x_hbm.at[pl.ds(off, _C)], idx_v)
        pltpu.async_copy(table_hbm.at[idx_v], rows_v, sem).wait()
        pltpu.sync_copy(rows_v, out_hbm.at[pl.ds(off, _C)])


def kernel(x, GloVe):
    xf = x.reshape(-1).astype(jnp.int32)
    table = jnp.pad(GloVe, ((0, 0), (0, _DP - _D)))
    out = _gather_kernel(xf, table)
    return out[:, :_D].reshape(x.shape + (GloVe.shape[1],))


# speedup vs baseline: 1.5317x; 1.5317x over previous
import functools

import jax
import jax.numpy as jnp
from jax import lax
from jax.experimental import pallas as pl
from jax.experimental.pallas import tpu as pltpu
from jax.experimental.pallas import tpu_sc as plsc

_D = 300
_B = 16384 * 200

_info = plsc.get_sparse_core_info()
_NW = _info.num_cores * _info.num_subcores
_RPW = _B // _NW
_C = 128
_STEPS = _RPW // _C

_mesh = plsc.VectorSubcoreMesh(core_axis_name="c", subcore_axis_name="s")


@functools.partial(
    pl.kernel,
    out_type=jax.ShapeDtypeStruct((_B, _D), jnp.float32),
    mesh=_mesh,
    scratch_types=[
        pltpu.VMEM((_C,), jnp.int32),
        pltpu.VMEM((_C, _D), jnp.float32),
        pltpu.VMEM((_C, 128), jnp.float32),
        pltpu.SemaphoreType.DMA,
    ],
    compiler_params=pltpu.CompilerParams(needs_layout_passes=False),
)
def _gather_kernel(x_hbm, t0_hbm, t1_hbm, t2_hbm, out_hbm,
                   idx_v, rows_v, tail_v, sem):
    wid = lax.axis_index("s") * _info.num_cores + lax.axis_index("c")
    base = wid * _RPW

    @pl.loop(0, _STEPS)
    def _step(i):
        off = base + i * _C
        pltpu.sync_copy(x_hbm.at[pl.ds(off, _C)], idx_v)
        c0 = pltpu.async_copy(t0_hbm.at[idx_v], rows_v.at[:, pl.ds(0, 128)], sem)
        c1 = pltpu.async_copy(t1_hbm.at[idx_v], rows_v.at[:, pl.ds(128, 128)], sem)
        c2 = pltpu.async_copy(t2_hbm.at[idx_v], tail_v, sem)
        c0.wait()
        c1.wait()
        c2.wait()

        col16 = jax.lax.iota(jnp.int32, 16) + 284

        @pl.loop(0, _C)
        def _patch(r):
            rows_v[r, pl.ds(256, 16)] = tail_v[r, pl.ds(84, 16)]
            rows_v[r, pl.ds(272, 16)] = tail_v[r, pl.ds(100, 16)]
            row16 = jnp.full((16,), r, dtype=jnp.int32)
            plsc.store_scatter(rows_v, [row16, col16], tail_v[r, pl.ds(112, 16)])

        pltpu.sync_copy(rows_v, out_hbm.at[pl.ds(off, _C)])


def kernel(x, GloVe):
    xf = x.reshape(-1).astype(jnp.int32)
    t0 = GloVe[:, 0:128]
    t1 = GloVe[:, 128:256]
    t2 = GloVe[:, 172:300]
    out = _gather_kernel(xf, t0, t1, t2)
    return out.reshape(x.shape + (GloVe.shape[1],))


# speedup vs baseline: 1.8811x; 1.2282x over previous
import functools

import jax
import jax.numpy as jnp
from jax import lax
from jax.experimental import pallas as pl
from jax.experimental.pallas import tpu as pltpu
from jax.experimental.pallas import tpu_sc as plsc

_D = 300
_B = 16384 * 200

_info = plsc.get_sparse_core_info()
_NW = _info.num_cores * _info.num_subcores
_RPW = _B // _NW
_C = 80
_STEPS = _RPW // _C

_mesh = plsc.VectorSubcoreMesh(core_axis_name="c", subcore_axis_name="s")


@functools.partial(
    pl.kernel,
    out_type=jax.ShapeDtypeStruct((_B, _D), jnp.float32),
    mesh=_mesh,
    scratch_types=[
        pltpu.VMEM((_C,), jnp.int32),
        pltpu.VMEM((_C,), jnp.int32),
        pltpu.VMEM((_C, _D), jnp.float32),
        pltpu.VMEM((_C, _D), jnp.float32),
        pltpu.VMEM((_C, 128), jnp.float32),
        pltpu.VMEM((_C, 128), jnp.float32),
        pltpu.SemaphoreType.DMA,
        pltpu.SemaphoreType.DMA,
    ],
    compiler_params=pltpu.CompilerParams(needs_layout_passes=False),
)
def _gather_kernel(x_hbm, t01_hbm, t2_hbm, out_hbm,
                   idx0, idx1, rows0, rows1, tail0, tail1, sem0, sem1):
    wid = lax.axis_index("s") * _info.num_cores + lax.axis_index("c")
    base = wid * _RPW
    col16 = lax.iota(jnp.int32, 16) + 284

    idx_v = (idx0, idx1)
    rows_v = (rows0, rows1)
    tail_v = (tail0, tail1)
    sems = (sem0, sem1)

    def issue(i, s):
        off = base + i * _C
        pltpu.sync_copy(x_hbm.at[pl.ds(off, _C)], idx_v[s])
        pltpu.async_copy(t01_hbm.at[idx_v[s]], rows_v[s].at[:, pl.ds(0, 256)],
                         sems[s])
        pltpu.async_copy(t2_hbm.at[idx_v[s]], tail_v[s], sems[s])

    def finish(i, s):
        pltpu.make_async_copy(t01_hbm.at[idx_v[s]],
                              rows_v[s].at[:, pl.ds(0, 256)], sems[s]).wait()
        pltpu.make_async_copy(t2_hbm.at[idx_v[s]], tail_v[s], sems[s]).wait()
        rv, tv = rows_v[s], tail_v[s]

        @pl.loop(0, _C)
        def _patch(r):
            rv[r, pl.ds(256, 16)] = tv[r, pl.ds(80, 16)]
            rv[r, pl.ds(272, 16)] = tv[r, pl.ds(96, 16)]
            row16 = jnp.full((16,), r, dtype=jnp.int32)
            plsc.store_scatter(rv, [row16, col16], tv[r, pl.ds(108, 16)])

        pltpu.sync_copy(rv, out_hbm.at[pl.ds(base + i * _C, _C)])

    issue(0, 0)

    @pl.loop(0, _STEPS, step=2)
    def _step(i):
        issue(i + 1, 1)
        finish(i, 0)

        @pl.when(i + 2 < _STEPS)
        def _():
            issue(i + 2, 0)

        finish(i + 1, 1)


def kernel(x, GloVe):
    xf = x.reshape(-1).astype(jnp.int32)
    t01 = GloVe[:, 0:256]
    t2 = jnp.pad(GloVe[:, 176:300], ((0, 0), (0, 4)))
    out = _gather_kernel(xf, t01, t2)
    return out.reshape(x.shape + (GloVe.shape[1],))


# speedup vs baseline: 1.9337x; 1.0279x over previous
import functools

import jax
import jax.numpy as jnp
from jax import lax
from jax.experimental import pallas as pl
from jax.experimental.pallas import tpu as pltpu
from jax.experimental.pallas import tpu_sc as plsc

_D = 300
_B = 16384 * 200

_info = plsc.get_sparse_core_info()
_NW = _info.num_cores * _info.num_subcores
_RPW = _B // _NW
_C = 80
_STEPS = _RPW // _C

_mesh = plsc.VectorSubcoreMesh(core_axis_name="c", subcore_axis_name="s")


@functools.partial(
    pl.kernel,
    out_type=jax.ShapeDtypeStruct((_B, _D), jnp.float32),
    mesh=_mesh,
    scratch_types=[
        pltpu.VMEM((_C,), jnp.int32),
        pltpu.VMEM((_C,), jnp.int32),
        pltpu.VMEM((_C, _D), jnp.float32),
        pltpu.VMEM((_C, _D), jnp.float32),
        pltpu.VMEM((_C, 128), jnp.float32),
        pltpu.VMEM((_C, 128), jnp.float32),
        pltpu.SemaphoreType.DMA,
        pltpu.SemaphoreType.DMA,
        pltpu.SemaphoreType.DMA,
        pltpu.SemaphoreType.DMA,
    ],
    compiler_params=pltpu.CompilerParams(needs_layout_passes=False),
)
def _gather_kernel(x_hbm, t01_hbm, t2_hbm, out_hbm,
                   idx0, idx1, rows0, rows1, tail0, tail1,
                   sem0, sem1, isem0, isem1):
    wid = lax.axis_index("s") * _info.num_cores + lax.axis_index("c")
    base = wid * _RPW
    col16 = lax.iota(jnp.int32, 16) + 284

    idx_v = (idx0, idx1)
    rows_v = (rows0, rows1)
    tail_v = (tail0, tail1)
    sems = (sem0, sem1)
    isems = (isem0, isem1)

    def issue_idx(i, s):
        pltpu.async_copy(x_hbm.at[pl.ds(base + i * _C, _C)], idx_v[s], isems[s])

    def issue_gathers(i, s):
        pltpu.make_async_copy(x_hbm.at[pl.ds(base + i * _C, _C)], idx_v[s],
                              isems[s]).wait()
        pltpu.async_copy(t01_hbm.at[idx_v[s]], rows_v[s].at[:, pl.ds(0, 256)],
                         sems[s])
        pltpu.async_copy(t2_hbm.at[idx_v[s]], tail_v[s], sems[s])

    def finish(i, s):
        pltpu.make_async_copy(t01_hbm.at[idx_v[s]],
                              rows_v[s].at[:, pl.ds(0, 256)], sems[s]).wait()
        pltpu.make_async_copy(t2_hbm.at[idx_v[s]], tail_v[s], sems[s]).wait()

        @pl.when(i + 2 < _STEPS)
        def _():
            issue_idx(i + 2, s)

        rv, tv = rows_v[s], tail_v[s]

        @pl.loop(0, _C)
        def _patch(r):
            rv[r, pl.ds(256, 16)] = tv[r, pl.ds(80, 16)]
            rv[r, pl.ds(272, 16)] = tv[r, pl.ds(96, 16)]
            row16 = jnp.full((16,), r, dtype=jnp.int32)
            plsc.store_scatter(rv, [row16, col16], tv[r, pl.ds(108, 16)])

        pltpu.sync_copy(rv, out_hbm.at[pl.ds(base + i * _C, _C)])

    issue_idx(0, 0)
    issue_idx(1, 1)
    issue_gathers(0, 0)

    @pl.loop(0, _STEPS, step=2)
    def _step(i):
        issue_gathers(i + 1, 1)
        finish(i, 0)

        @pl.when(i + 2 < _STEPS)
        def _():
            issue_gathers(i + 2, 0)

        finish(i + 1, 1)


def kernel(x, GloVe):
    xf = x.reshape(-1).astype(jnp.int32)
    t01 = GloVe[:, 0:256]
    t2 = jnp.pad(GloVe[:, 176:300], ((0, 0), (0, 4)))
    out = _gather_kernel(xf, t01, t2)
    return out.reshape(x.shape + (GloVe.shape[1],))
